# baseline (speedup 1.0000x reference)
import dataclasses

import jax
import jax.numpy as jnp
from jax.experimental import pallas as pl
from jax.experimental.pallas import tpu as pltpu
from jax.experimental.pallas import tpu_sc as plsc

_SC_COMPILER_PARAMS = pltpu.CompilerParams()
if "needs_layout_passes" in pltpu.CompilerParams.__dataclass_fields__:
    _SC_COMPILER_PARAMS = dataclasses.replace(
        _SC_COMPILER_PARAMS, needs_layout_passes=False)

_TOKENS = 16384
_DIM = 4096
_EXPERTS = 64
_BLOCK = 1024
_SLABS = 1
_SLAB = _TOKENS // _SLABS
_LANES = 16
_TILE = 256


def _logits_kernel(x_ref, w_ref, m_ref, logits_ref, scores_t_ref):
    x = x_ref[...]
    w = w_ref[...]
    mask = m_ref[...]
    logits = jax.lax.dot_general(
        x, w, (((1,), (1,)), ((), ())), preferred_element_type=jnp.float32)
    logits = jnp.where(mask == 0.0, -1000000000.0, logits)
    logits_ref[...] = logits
    row_max = jnp.max(logits, axis=-1, keepdims=True)
    e = jnp.exp(logits - row_max)
    s = e / jnp.sum(e, axis=-1, keepdims=True) + 1e-14
    scores_t_ref[...] = s.T


def _tc_logits(x, W, mask2d, slab):
    grid = (_SLAB // _BLOCK,)
    base = slab * (_SLAB // _BLOCK)
    return pl.pallas_call(
        _logits_kernel,
        grid=grid,
        in_specs=[
            pl.BlockSpec((_BLOCK, _DIM), lambda i: (base + i, 0)),
            pl.BlockSpec((_EXPERTS, _DIM), lambda i: (0, 0)),
            pl.BlockSpec((1, _EXPERTS), lambda i: (0, 0)),
        ],
        out_specs=[
            pl.BlockSpec((_BLOCK, _EXPERTS), lambda i: (i, 0)),
            pl.BlockSpec((_EXPERTS, _BLOCK), lambda i: (0, i)),
        ],
        out_shape=[
            jax.ShapeDtypeStruct((_SLAB, _EXPERTS), jnp.float32),
            jax.ShapeDtypeStruct((_EXPERTS, _SLAB), jnp.float32),
        ],
        compiler_params=pltpu.CompilerParams(
            dimension_semantics=("parallel",)),
    )(x, W, mask2d)


def _sc_topk(scores_t, mask2d):
    s_tokens = scores_t.shape[1]
    mesh = plsc.VectorSubcoreMesh(
        core_axis_name="core", subcore_axis_name="subcore")

    @pl.kernel(
        out_type=jax.ShapeDtypeStruct((s_tokens,), jnp.int32),
        mesh=mesh,
        scratch_types=[
            pltpu.VMEM((1, _EXPERTS), jnp.float32),
            pltpu.SemaphoreType.DMA,
        ],
        compiler_params=_SC_COMPILER_PARAMS,
    )
    def sc_kernel(st_hbm, mask_hbm, o_hbm, mask_vmem, sem):
        pltpu.async_copy(mask_hbm, mask_vmem, sem).wait()

        def body(st_vmem, o_vmem):
            act_v = None
            for c4 in range(_EXPERTS // _LANES):
                mc = mask_vmem[0, pl.ds(c4 * _LANES, _LANES)]
                av = jnp.where(mc != 0.0, jnp.int32(1), jnp.int32(0))
                act_v = av if act_v is None else act_v + av
            active = jnp.sum(act_v)
            for c in range(0, _TILE // _LANES, 2):
                sl0 = pl.ds(c * _LANES, _LANES)
                sl1 = pl.ds((c + 1) * _LANES, _LANES)
                tp0 = [st_vmem[e, sl0] for e in range(4)]
                tp1 = [st_vmem[e, sl1] for e in range(4)]
                for e in range(4, _EXPERTS):
                    tp0[e % 4] = tp0[e % 4] + st_vmem[e, sl0]
                    tp1[e % 4] = tp1[e % 4] + st_vmem[e, sl1]
                tau0 = ((tp0[0] + tp0[1]) + (tp0[2] + tp0[3])) - 1.0
                tau1 = ((tp1[0] + tp1[1]) + (tp1[2] + tp1[3])) - 1.0
                cp0 = [None] * 4
                cp1 = [None] * 4
                for e in range(_EXPERTS):
                    cb0 = jnp.where(st_vmem[e, sl0] > tau0,
                                    jnp.int32(1), jnp.int32(0))
                    cb1 = jnp.where(st_vmem[e, sl1] > tau1,
                                    jnp.int32(1), jnp.int32(0))
                    cp0[e % 4] = cb0 if cp0[e % 4] is None else cp0[e % 4] + cb0
                    cp1[e % 4] = cb1 if cp1[e % 4] is None else cp1[e % 4] + cb1
                cnt0 = (cp0[0] + cp0[1]) + (cp0[2] + cp0[3])
                cnt1 = (cp1[0] + cp1[1]) + (cp1[2] + cp1[3])
                o_vmem[sl0] = jnp.minimum(cnt0, active)
                o_vmem[sl1] = jnp.minimum(cnt1, active)

        pltpu.emit_pipeline(
            body,
            grid=(s_tokens // _TILE,),
            in_specs=[pl.BlockSpec((_EXPERTS, _TILE), lambda i: (0, i))],
            out_specs=[pl.BlockSpec((_TILE,), lambda i: (i,))],
            core_axis_name=("core", "subcore"),
            dimension_semantics=(pltpu.PARALLEL,),
        )(st_hbm, o_hbm)

    return sc_kernel(scores_t, mask2d)


def kernel(x, W, experts_mask):
    mask2d = experts_mask.reshape(1, _EXPERTS)
    logits, scores_t = _tc_logits(x, W, mask2d, 0)
    topk = _sc_topk(scores_t, mask2d)
    return (logits, topk)


# speedup vs baseline: 1.0316x; 1.0316x over previous
import dataclasses

import jax
import jax.numpy as jnp
from jax.experimental import pallas as pl
from jax.experimental.pallas import tpu as pltpu
from jax.experimental.pallas import tpu_sc as plsc

_SC_COMPILER_PARAMS = pltpu.CompilerParams()
if "needs_layout_passes" in pltpu.CompilerParams.__dataclass_fields__:
    _SC_COMPILER_PARAMS = dataclasses.replace(
        _SC_COMPILER_PARAMS, needs_layout_passes=False)

_TOKENS = 16384
_DIM = 4096
_EXPERTS = 64
_BLOCK = 1024
_SLABS = 1
_SLAB = _TOKENS // _SLABS
_LANES = 16
_TILE = 128


def _logits_kernel(x_ref, w_ref, m_ref, logits_ref, scores_t_ref):
    x = x_ref[...]
    w = w_ref[...]
    mask = m_ref[...]
    logits = jax.lax.dot_general(
        x, w, (((1,), (1,)), ((), ())), preferred_element_type=jnp.float32)
    logits = jnp.where(mask == 0.0, -1000000000.0, logits)
    logits_ref[...] = logits
    row_max = jnp.max(logits, axis=-1, keepdims=True)
    e = jnp.exp(logits - row_max)
    s = e / jnp.sum(e, axis=-1, keepdims=True) + 1e-14
    scores_t_ref[...] = s.T


def _tc_logits(x, W, mask2d, slab):
    grid = (_SLAB // _BLOCK,)
    base = slab * (_SLAB // _BLOCK)
    return pl.pallas_call(
        _logits_kernel,
        grid=grid,
        in_specs=[
            pl.BlockSpec((_BLOCK, _DIM), lambda i: (base + i, 0)),
            pl.BlockSpec((_EXPERTS, _DIM), lambda i: (0, 0)),
            pl.BlockSpec((1, _EXPERTS), lambda i: (0, 0)),
        ],
        out_specs=[
            pl.BlockSpec((_BLOCK, _EXPERTS), lambda i: (i, 0)),
            pl.BlockSpec((_EXPERTS, _BLOCK), lambda i: (0, i)),
        ],
        out_shape=[
            jax.ShapeDtypeStruct((_SLAB, _EXPERTS), jnp.float32),
            jax.ShapeDtypeStruct((_EXPERTS, _SLAB), jnp.float32),
        ],
        compiler_params=pltpu.CompilerParams(
            dimension_semantics=("parallel",)),
    )(x, W, mask2d)


def _sc_topk(scores_t, mask2d):
    s_tokens = scores_t.shape[1]
    mesh = plsc.VectorSubcoreMesh(
        core_axis_name="core", subcore_axis_name="subcore")

    @pl.kernel(
        out_type=jax.ShapeDtypeStruct((s_tokens,), jnp.int32),
        mesh=mesh,
        scratch_types=[
            pltpu.VMEM((1, _EXPERTS), jnp.float32),
            pltpu.SemaphoreType.DMA,
        ],
        compiler_params=_SC_COMPILER_PARAMS,
    )
    def sc_kernel(st_hbm, mask_hbm, o_hbm, mask_vmem, sem):
        pltpu.async_copy(mask_hbm, mask_vmem, sem).wait()

        def body(st_vmem, o_vmem):
            act_v = None
            for c4 in range(_EXPERTS // _LANES):
                mc = mask_vmem[0, pl.ds(c4 * _LANES, _LANES)]
                av = jnp.where(mc != 0.0, jnp.int32(1), jnp.int32(0))
                act_v = av if act_v is None else act_v + av
            active = jnp.sum(act_v)
            for c in range(0, _TILE // _LANES, 2):
                sl0 = pl.ds(c * _LANES, _LANES)
                sl1 = pl.ds((c + 1) * _LANES, _LANES)
                tp0 = [st_vmem[e, sl0] for e in range(4)]
                tp1 = [st_vmem[e, sl1] for e in range(4)]
                for e in range(4, _EXPERTS):
                    tp0[e % 4] = tp0[e % 4] + st_vmem[e, sl0]
                    tp1[e % 4] = tp1[e % 4] + st_vmem[e, sl1]
                tau0 = ((tp0[0] + tp0[1]) + (tp0[2] + tp0[3])) - 1.0
                tau1 = ((tp1[0] + tp1[1]) + (tp1[2] + tp1[3])) - 1.0
                cp0 = [None] * 4
                cp1 = [None] * 4
                for e in range(_EXPERTS):
                    cb0 = jnp.where(st_vmem[e, sl0] > tau0,
                                    jnp.int32(1), jnp.int32(0))
                    cb1 = jnp.where(st_vmem[e, sl1] > tau1,
                                    jnp.int32(1), jnp.int32(0))
                    cp0[e % 4] = cb0 if cp0[e % 4] is None else cp0[e % 4] + cb0
                    cp1[e % 4] = cb1 if cp1[e % 4] is None else cp1[e % 4] + cb1
                cnt0 = (cp0[0] + cp0[1]) + (cp0[2] + cp0[3])
                cnt1 = (cp1[0] + cp1[1]) + (cp1[2] + cp1[3])
                o_vmem[sl0] = jnp.minimum(cnt0, active)
                o_vmem[sl1] = jnp.minimum(cnt1, active)

        pltpu.emit_pipeline(
            body,
            grid=(s_tokens // _TILE,),
            in_specs=[pl.BlockSpec((_EXPERTS, _TILE), lambda i: (0, i))],
            out_specs=[pl.BlockSpec((_TILE,), lambda i: (i,))],
            core_axis_name=("core", "subcore"),
            dimension_semantics=(pltpu.PARALLEL,),
        )(st_hbm, o_hbm)

    return sc_kernel(scores_t, mask2d)


def kernel(x, W, experts_mask):
    mask2d = experts_mask.reshape(1, _EXPERTS)
    logits, scores_t = _tc_logits(x, W, mask2d, 0)
    topk = _sc_topk(scores_t, mask2d)
    return (logits, topk)
